# baseline (speedup 1.0000x reference)
import functools

import jax
import jax.numpy as jnp
from jax import lax
from jax.experimental import pallas as pl
from jax.experimental.pallas import tpu as pltpu
from jax.experimental.pallas import tpu_sc as plsc

VOCAB = 1000
D = 128
BATCH = 4096
SEQ = 200
N = BATCH * SEQ

NC = 2
NS = 16
NW = NC * NS
CHUNK = 128
NBUF = 5
STAGERS = 5
VPS = VOCAB // STAGERS


def _make_lookup(chunks):
    rpw = chunks * CHUNK
    groups = chunks // NBUF

    @functools.partial(
        pl.kernel,
        out_type=jax.ShapeDtypeStruct((NW * rpw, D), jnp.float32),
        mesh=plsc.VectorSubcoreMesh(core_axis_name="c", subcore_axis_name="s"),
        scratch_types=[
            pltpu.VMEM((chunks, CHUNK), jnp.int32),
            pltpu.VMEM((NBUF, CHUNK, D), jnp.float32),
            pltpu.VMEM_SHARED((VOCAB, D), jnp.float32),
            pltpu.SemaphoreType.DMA((NBUF,)),
            pltpu.SemaphoreType.DMA((NBUF,)),
        ],
    )
    def _emb_lookup(table_hbm, idx_hbm, out_hbm, idx_v, rows_v,
                    table_spm, gsem, ssem):
        sid = lax.axis_index("s")
        wid = sid * NC + lax.axis_index("c")

        @pl.when(sid < STAGERS)
        def _stage():
            for p, (off, rows) in enumerate(((0, CHUNK), (CHUNK, VPS - CHUNK))):
                base = sid * VPS + off
                pltpu.sync_copy(table_hbm.at[pl.ds(base, rows)],
                                rows_v.at[p].at[pl.ds(0, rows)])
                pltpu.sync_copy(rows_v.at[p].at[pl.ds(0, rows)],
                                table_spm.at[pl.ds(base, rows)])

        pltpu.sync_copy(idx_hbm.at[wid], idx_v)
        plsc.subcore_barrier()
        out_base = wid * rpw

        def gather(j, b):
            pltpu.async_copy(table_spm.at[idx_v.at[j]], rows_v.at[b],
                             gsem.at[b])

        def store(j, b):
            pltpu.async_copy(
                rows_v.at[b], out_hbm.at[pl.ds(out_base + j * CHUNK, CHUNK)],
                ssem.at[b])

        def wait(sem, b, rows=CHUNK):
            pltpu.make_async_copy(
                table_hbm.at[pl.ds(0, rows)],
                rows_v.at[b].at[pl.ds(0, rows)], sem.at[b]).wait()

        for b in range(NBUF):
            gather(b, b)

        def body(i, carry):
            for b in range(NBUF):
                j = i * NBUF + b
                wait(gsem, b)
                store(j, b)
                wait(ssem, b)
                gather(j + NBUF, b)
            return carry

        lax.fori_loop(0, groups - 1, body, 0)

        for b in range(NBUF):
            j = (groups - 1) * NBUF + b
            wait(gsem, b)
            store(j, b)
        for b in range(NBUF):
            wait(ssem, b)

    return _emb_lookup


_CHUNKS = N // (NW * CHUNK)
_LOOKUP = _make_lookup(_CHUNKS)


def kernel(input_seq, embedding_weight):
    idx = input_seq.reshape(NW, _CHUNKS, CHUNK).astype(jnp.int32)
    out = _LOOKUP(embedding_weight, idx)
    return out.reshape(BATCH, SEQ, D)


# speedup vs baseline: 1.0023x; 1.0023x over previous
import functools

import jax
import jax.numpy as jnp
from jax import lax
from jax.experimental import pallas as pl
from jax.experimental.pallas import tpu as pltpu
from jax.experimental.pallas import tpu_sc as plsc

VOCAB = 1000
D = 128
BATCH = 4096
SEQ = 200
N = BATCH * SEQ

NC = 2
NS = 16
NW = NC * NS
CHUNK = 128
NBUF = 4
STAGERS = 5
VPS = VOCAB // STAGERS


def _make_lookup(chunks):
    rpw = chunks * CHUNK
    groups = chunks // NBUF

    @functools.partial(
        pl.kernel,
        out_type=jax.ShapeDtypeStruct((NW * rpw, D), jnp.float32),
        mesh=plsc.VectorSubcoreMesh(core_axis_name="c", subcore_axis_name="s"),
        scratch_types=[
            pltpu.VMEM((chunks, CHUNK), jnp.int32),
            pltpu.VMEM((NBUF, CHUNK, D), jnp.float32),
            pltpu.VMEM_SHARED((VOCAB, D), jnp.float32),
            pltpu.SemaphoreType.DMA((NBUF,)),
            pltpu.SemaphoreType.DMA((NBUF,)),
        ],
    )
    def _emb_lookup(table_hbm, idx_hbm, out_hbm, idx_v, rows_v,
                    table_spm, gsem, ssem):
        sid = lax.axis_index("s")
        wid = sid * NC + lax.axis_index("c")

        @pl.when(sid < STAGERS)
        def _stage():
            for p, (off, rows) in enumerate(((0, CHUNK), (CHUNK, VPS - CHUNK))):
                base = sid * VPS + off
                pltpu.sync_copy(table_hbm.at[pl.ds(base, rows)],
                                rows_v.at[p].at[pl.ds(0, rows)])
                pltpu.sync_copy(rows_v.at[p].at[pl.ds(0, rows)],
                                table_spm.at[pl.ds(base, rows)])

        pltpu.sync_copy(idx_hbm.at[wid], idx_v)
        plsc.subcore_barrier()
        out_base = wid * rpw

        def gather(j, b):
            pltpu.async_copy(table_spm.at[idx_v.at[j]], rows_v.at[b],
                             gsem.at[b])

        def store(j, b):
            pltpu.async_copy(
                rows_v.at[b], out_hbm.at[pl.ds(out_base + j * CHUNK, CHUNK)],
                ssem.at[b])

        def wait(sem, b, rows=CHUNK):
            pltpu.make_async_copy(
                table_hbm.at[pl.ds(0, rows)],
                rows_v.at[b].at[pl.ds(0, rows)], sem.at[b]).wait()

        for b in range(NBUF):
            gather(b, b)

        def body(i, carry):
            for b in range(NBUF):
                j = i * NBUF + b
                wait(gsem, b)
                store(j, b)
                wait(ssem, b)
                gather(j + NBUF, b)
            return carry

        lax.fori_loop(0, groups - 1, body, 0)

        for b in range(NBUF):
            j = (groups - 1) * NBUF + b
            wait(gsem, b)
            store(j, b)
        for b in range(NBUF):
            wait(ssem, b)

    return _emb_lookup


_CHUNKS = N // (NW * CHUNK)
_LOOKUP = _make_lookup(_CHUNKS)


def kernel(input_seq, embedding_weight):
    idx = input_seq.reshape(NW, _CHUNKS, CHUNK).astype(jnp.int32)
    out = _LOOKUP(embedding_weight, idx)
    return out.reshape(BATCH, SEQ, D)
